# baseline (speedup 1.0000x reference)
import functools

import jax
import jax.numpy as jnp
from jax import lax
from jax.experimental import pallas as pl
from jax.experimental.pallas import tpu as pltpu
from jax.experimental.pallas import tpu_sc as plsc

_NC = 2
_NS = 16
_NW = _NC * _NS
_LANES = 16


@functools.cache
def _build(B, C, H, W, interpret=False):
    HW = H * W
    P = B * HW
    assert P % _NW == 0
    per_w = P // _NW
    assert HW % per_w == 0, "each worker must stay within one batch"
    wpb = HW // per_w
    BLK = min(2048, per_w)
    assert per_w % BLK == 0 and BLK % 128 == 0
    NBLK = per_w // BLK
    NI = BLK // 128
    boff_unit = (C - 1) * HW

    mesh = plsc.VectorSubcoreMesh(
        core_axis_name="c", subcore_axis_name="s",
        num_cores=_NC, num_subcores=_NS)

    @functools.partial(
        pl.kernel,
        out_type=jax.ShapeDtypeStruct((_NW, _LANES), jnp.float32),
        mesh=mesh,
        scratch_types=[
            pltpu.VMEM((BLK,), jnp.int32),
            pltpu.VMEM((BLK,), jnp.float32),
            pltpu.VMEM((NI, 128), jnp.int32),
            pltpu.VMEM((NI, 128), jnp.float32),
            pltpu.VMEM((_LANES,), jnp.float32),
            pltpu.SemaphoreType.DMA,
        ],
        interpret=interpret,
    )
    def sc_kernel(z_hbm, l_hbm, c_hbm, out_hbm, l_v, c_v, idx_v, val_v,
                  acc_v, sem):
        wid = lax.axis_index("s") * _NC + lax.axis_index("c")
        base = wid * per_w
        b = wid // wpb
        boff = b * boff_unit
        iota = lax.iota(jnp.int32, _LANES)

        def block(i, acc):
            bb = base + i * BLK
            pltpu.sync_copy(l_hbm.at[pl.ds(bb, BLK)], l_v)
            pltpu.sync_copy(c_hbm.at[pl.ds(bb, BLK)], c_v)

            def ix(j, carry):
                lv = l_v[pl.ds(j * _LANES, _LANES)]
                pix = bb + boff + j * _LANES
                idx_v[j // 8, pl.ds((j % 8) * _LANES, _LANES)] = (
                    lv * HW + (iota + pix))
                return carry

            lax.fori_loop(0, BLK // _LANES, ix, 0, unroll=4)

            cps = [pltpu.async_copy(z_hbm.at[idx_v.at[r]], val_v.at[r], sem)
                   for r in range(NI)]
            for cp in cps:
                cp.wait()

            def ac(j, a):
                v = val_v[j // 8, pl.ds((j % 8) * _LANES, _LANES)]
                m = c_v[pl.ds(j * _LANES, _LANES)]
                return a + v * m

            return lax.fori_loop(0, BLK // _LANES, ac, acc, unroll=4)

        acc = lax.fori_loop(0, NBLK, block, jnp.zeros((_LANES,), jnp.float32))
        acc_v[...] = acc
        pltpu.sync_copy(acc_v, out_hbm.at[wid])

    return sc_kernel


def kernel(z, condition, l):
    B, C, H, W = z.shape
    zf = z.reshape(-1)
    lf = l.astype(jnp.int32).reshape(-1)
    cf = condition.reshape(-1).astype(jnp.float32)
    partials = _build(B, C, H, W)(zf, lf, cf)
    return jnp.sum(partials)
